# Initial kernel scaffold; baseline (speedup 1.0000x reference)
import functools

import jax
import jax.numpy as jnp
from jax import lax
from jax.experimental import pallas as pl
from jax.experimental.pallas import tpu as pltpu
from jax.experimental.pallas import tpu_sc as plsc

B = 16384
F = 26
W = 32
NC = 2
NS = 16
NW = NC * NS
BPW = B // NW
C = 4
IPC = C * F
NCHUNK = BPW // C
HALF = W // 2

_mesh = plsc.VectorSubcoreMesh(core_axis_name="c", subcore_axis_name="s")


@functools.partial(
    pl.kernel,
    mesh=_mesh,
    out_type=jax.ShapeDtypeStruct((B, W), jnp.float32),
    scratch_types=[
        pltpu.VMEM((NCHUNK, IPC), jnp.int32),
        pltpu.VMEM((2, IPC, W), jnp.float32),
        pltpu.VMEM((BPW, W), jnp.float32),
        pltpu.SemaphoreType.DMA,
        pltpu.SemaphoreType.DMA,
    ],
)
def _embed_sum(x_hbm, emb_hbm, out_hbm, idx_v, rows_v, out_v, sem_a, sem_b):
    wid = lax.axis_index("c") * NS + lax.axis_index("s")

    pltpu.sync_copy(x_hbm.at[wid], idx_v)

    sems = (sem_a, sem_b)

    def issue(chunk, buf):
        pltpu.async_copy(emb_hbm.at[idx_v.at[chunk]], rows_v.at[buf], sems[buf])

    def wait(buf):
        pltpu.make_async_copy(
            emb_hbm.at[pl.ds(0, IPC)], rows_v.at[buf], sems[buf]
        ).wait()

    def compute(chunk, buf):
        for e in range(C):
            r0 = e * F
            acc_lo = rows_v[buf, r0, pl.ds(0, HALF)]
            acc_hi = rows_v[buf, r0, pl.ds(HALF, HALF)]
            for f in range(1, F):
                acc_lo = acc_lo + rows_v[buf, r0 + f, pl.ds(0, HALF)]
                acc_hi = acc_hi + rows_v[buf, r0 + f, pl.ds(HALF, HALF)]
            row = chunk * C + e
            out_v[row, pl.ds(0, HALF)] = acc_lo
            out_v[row, pl.ds(HALF, HALF)] = acc_hi

    issue(0, 0)

    def body(i, carry):
        g = 2 * i
        issue(g + 1, 1)
        wait(0)
        compute(g, 0)

        @pl.when(g + 2 < NCHUNK)
        def _():
            issue(g + 2, 0)

        wait(1)
        compute(g + 1, 1)
        return carry

    lax.fori_loop(0, NCHUNK // 2, body, 0)

    pltpu.sync_copy(out_v, out_hbm.at[pl.ds(wid * BPW, BPW)])


def kernel(x, embeddings):
    x = x.astype(jnp.int32).reshape(NW, NCHUNK, IPC)
    return _embed_sum(x, embeddings)


# speedup vs baseline: 1.9223x; 1.9223x over previous
import functools

import jax
import jax.numpy as jnp
from jax import lax
from jax.experimental import pallas as pl
from jax.experimental.pallas import tpu as pltpu
from jax.experimental.pallas import tpu_sc as plsc

B = 16384
F = 26
W = 32
NC = 2
NS = 16
NW = NC * NS
BPW = B // NW
C = 16
IPC = C * F
NCHUNK = BPW // C
HALF = W // 2

_mesh = plsc.VectorSubcoreMesh(core_axis_name="c", subcore_axis_name="s")


@functools.partial(
    pl.kernel,
    mesh=_mesh,
    compiler_params=pltpu.CompilerParams(use_tc_tiling_on_sc=False),
    out_type=jax.ShapeDtypeStruct((B, W), jnp.float32),
    scratch_types=[
        pltpu.VMEM((NCHUNK, IPC), jnp.int32),
        pltpu.VMEM((2, IPC, W), jnp.float32),
        pltpu.VMEM((BPW, W), jnp.float32),
        pltpu.SemaphoreType.DMA,
        pltpu.SemaphoreType.DMA,
    ],
)
def _embed_sum(x_hbm, emb_hbm, out_hbm, idx_v, rows_v, out_v, sem_a, sem_b):
    wid = lax.axis_index("c") * NS + lax.axis_index("s")

    pltpu.sync_copy(x_hbm.at[wid], idx_v)

    sems = (sem_a, sem_b)

    def issue(chunk, buf):
        pltpu.async_copy(emb_hbm.at[idx_v.at[chunk]], rows_v.at[buf], sems[buf])

    def wait(buf):
        pltpu.make_async_copy(
            emb_hbm.at[pl.ds(0, IPC)], rows_v.at[buf], sems[buf]
        ).wait()

    def compute(chunk, buf):
        for e in range(C):
            r0 = e * F
            acc_lo = rows_v[buf, r0, pl.ds(0, HALF)]
            acc_hi = rows_v[buf, r0, pl.ds(HALF, HALF)]
            for f in range(1, F):
                acc_lo = acc_lo + rows_v[buf, r0 + f, pl.ds(0, HALF)]
                acc_hi = acc_hi + rows_v[buf, r0 + f, pl.ds(HALF, HALF)]
            row = chunk * C + e
            out_v[row, pl.ds(0, HALF)] = acc_lo
            out_v[row, pl.ds(HALF, HALF)] = acc_hi

    issue(0, 0)

    def body(i, carry):
        g = 2 * i
        issue(g + 1, 1)
        wait(0)
        compute(g, 0)

        @pl.when(g + 2 < NCHUNK)
        def _():
            issue(g + 2, 0)

        wait(1)
        compute(g + 1, 1)
        return carry

    lax.fori_loop(0, NCHUNK // 2, body, 0)

    pltpu.sync_copy(out_v, out_hbm.at[pl.ds(wid * BPW, BPW)])


def kernel(x, embeddings):
    x = x.astype(jnp.int32).reshape(NW, NCHUNK, IPC)
    return _embed_sum(x, embeddings)
